# baseline (speedup 1.0000x reference)
import functools

import jax
import jax.numpy as jnp
from jax import lax
from jax.experimental import pallas as pl
from jax.experimental.pallas import tpu as pltpu
from jax.experimental.pallas import tpu_sc as plsc

H, B, V, D = 200, 4096, 1000000, 32
NW = 32
UNITS = (H // 8) * (B // 128)
UNITS_W = UNITS // NW
NBUF = 3


def _rsqrt_newton(ss):
    xhalf = 0.5 * ss
    i = lax.bitcast_convert_type(ss, jnp.int32)
    i = jnp.int32(0x5F3759DF) - (i >> 1)
    y = lax.bitcast_convert_type(i, jnp.float32)
    y = y * (1.5 - xhalf * y * y)
    y = y * (1.5 - xhalf * y * y)
    y = y * (1.5 - xhalf * y * y)
    return y


def _make_sc_call():
    mesh_sc = plsc.VectorSubcoreMesh(core_axis_name="c", subcore_axis_name="s")

    @functools.partial(
        pl.kernel,
        out_type=jax.ShapeDtypeStruct((H, 4, B // 128, 8, 128), jnp.float32),
        mesh=mesh_sc,
        scratch_types=[
            pltpu.VMEM((NBUF * 8, 128), jnp.int32),
            pltpu.VMEM((NBUF * 1024, 32), jnp.float32),
            pltpu.VMEM((64, 128), jnp.float32),
            pltpu.SemaphoreType.DMA,
            pltpu.SemaphoreType.DMA,
            pltpu.SemaphoreType.DMA,
            pltpu.SemaphoreType.DMA,
            pltpu.SemaphoreType.DMA,
            pltpu.SemaphoreType.DMA,
        ],
        compiler_params=pltpu.CompilerParams(
            needs_layout_passes=False, use_tc_tiling_on_sc=False
        ),
    )
    def sck(
        w_hbm, a2_hbm, out_hbm, idx_v, grows_v, outb_v,
        gsem0, gsem1, gsem2, isem, osem0, osem1,
    ):
        gsems = (gsem0, gsem1, gsem2)
        osems = (osem0, osem1)
        wid = lax.axis_index("s") * 2 + lax.axis_index("c")
        iota = lax.iota(jnp.int32, 16)
        u0 = wid * UNITS_W

        def stage_idx(u, par):
            return pltpu.async_copy(
                a2_hbm.at[pl.ds(pl.multiple_of((u0 + u) * 8, 8), 8)],
                idx_v.at[pl.ds(pl.multiple_of(par * 8, 8), 8)],
                isem,
            )

        def drain_idx(par):
            pltpu.make_async_copy(
                a2_hbm.at[pl.ds(0, 8)],
                idx_v.at[pl.ds(pl.multiple_of(par * 8, 8), 8)],
                isem,
            ).wait()

        def fire_gathers(par, sem):
            for hl in range(8):
                pltpu.async_copy(
                    w_hbm.at[idx_v.at[par * 8 + hl]],
                    grows_v.at[
                        pl.ds(pl.multiple_of(par * 1024 + hl * 128, 128), 128),
                        :,
                    ],
                    sem,
                )

        def drain_gathers(par, sem):
            for hl in range(8):
                pltpu.make_async_copy(
                    w_hbm.at[idx_v.at[par * 8 + hl]],
                    grows_v.at[
                        pl.ds(pl.multiple_of(par * 1024 + hl * 128, 128), 128),
                        :,
                    ],
                    sem,
                ).wait()

        def gsem_dispatch(par, fn):
            for k in range(NBUF):
                @pl.when(par == k)
                def _(k=k):
                    fn(k, gsems[k])

        def drain_out(h_par):
            for g in range(4):
                pltpu.make_async_copy(
                    outb_v.at[pl.ds(h_par * 32 + g * 8, 8), :],
                    out_hbm.at[0, g, 0],
                    osems[h_par],
                ).wait()

        stage_idx(0, 0).wait()
        fire_gathers(0, gsems[0])
        stage_idx(1, 1).wait()
        fire_gathers(1, gsems[1])
        stage_idx(2, 2)

        def do_unit(u, carry):
            unit = u0 + u
            h8 = unit // (B // 128)
            b1 = unit % (B // 128)
            par = lax.rem(u, NBUF)

            gsem_dispatch(par, drain_gathers)

            @pl.when(u < UNITS_W - 2)
            def _():
                par2 = lax.rem(u + 2, NBUF)
                drain_idx(par2)
                gsem_dispatch(par2, fire_gathers)

            @pl.when(u < UNITS_W - 3)
            def _():
                stage_idx(u + 3, par)

            for hl in range(8):
                hp = hl & 1
                if hl >= 2:
                    drain_out(hp)
                else:
                    @pl.when(u > 0)
                    def _():
                        drain_out(hp)

                def do_block(j, inner):
                    row_idx = par * 1024 + hl * 128 + j * 16 + iota
                    colb = j * 16 + iota
                    cs = [(iota + k) & 31 for k in range(4)]
                    sss = [jnp.zeros((16,), jnp.float32) for _ in range(4)]
                    for _t in range(8):
                        for k in range(4):
                            g = plsc.load_gather(grows_v, [row_idx, cs[k]])
                            sss[k] = sss[k] + g * g
                            cs[k] = (cs[k] + 4) & 31
                    ss = (sss[0] + sss[1]) + (sss[2] + sss[3])
                    y = _rsqrt_newton(jnp.maximum(ss, 1e-24))
                    cs = [(iota + k) & 31 for k in range(4)]
                    for _t in range(8):
                        for k in range(4):
                            g = plsc.load_gather(grows_v, [row_idx, cs[k]])
                            plsc.store_scatter(
                                outb_v, [hp * 32 + cs[k], colb], g * y
                            )
                            cs[k] = (cs[k] + 4) & 31
                    return inner

                lax.fori_loop(0, 8, do_block, 0)

                h = h8 * 8 + hl
                for g in range(4):
                    pltpu.async_copy(
                        outb_v.at[pl.ds(hp * 32 + g * 8, 8), :],
                        out_hbm.at[h, g, b1],
                        osems[hp],
                    )
            return carry

        lax.fori_loop(0, UNITS_W, do_unit, 0)
        drain_out(0)
        drain_out(1)

    return sck


def kernel(input, W):
    a2 = (
        jnp.transpose(input, (1, 0))
        .astype(jnp.int32)
        .reshape(25, 8, 32, 128)
        .transpose(0, 2, 1, 3)
        .reshape(6400, 128)
    )
    o5 = _make_sc_call()(W, a2)
    return jnp.transpose(o5, (0, 2, 4, 1, 3)).reshape(H, B, D)
